# baseline (speedup 1.0000x reference)
import jax
import jax.numpy as jnp
from jax.experimental import pallas as pl
from jax.experimental.pallas import tpu as pltpu

K = 16384
C = 1000
ALPHA = 0.1
SIGMA = 0.01

LOGN = 14
CH = 1024
NCH = K // CH
NBUF = 4


def _fused_body(lab_ref, w_ref, smx_ref, out_ref, r_sc, *bufs_and_sems):
    bufs = bufs_and_sems[:NBUF]
    sems = bufs_and_sems[NBUF:]

    def dma(ci):
        return pltpu.make_async_copy(
            smx_ref.at[pl.ds(ci * CH, CH), :], bufs[ci % NBUF], sems[ci % NBUF])

    for ci in range(NBUF):
        dma(ci).start()

    ii = jax.lax.broadcasted_iota(jnp.int32, (128, 128), 0) * 128 + \
        jax.lax.broadcasted_iota(jnp.int32, (128, 128), 1)
    eye = jnp.where(
        jax.lax.broadcasted_iota(jnp.int32, (128, 128), 0)
        == jax.lax.broadcasted_iota(jnp.int32, (128, 128), 1), 1.0, 0.0
    ).astype(jnp.float32)
    colf = jax.lax.broadcasted_iota(jnp.int32, (128, C), 1).astype(jnp.float32)

    RG = CH // 128
    for ci in range(NCH):
        dma(ci).wait()
        if ci + NBUF < NCH:
            dma(ci + NBUF).start()
        for g in range(RG):
            lrf = lab_ref[pl.ds(ci * RG + g, 1), :].astype(jnp.float32)
            lab_col = jax.lax.dot_general(
                eye, lrf, (((1,), (1,)), ((), ())),
                preferred_element_type=jnp.float32)
            chunk = bufs[ci % NBUF][pl.ds(g * 128, 128), :]
            masked = jnp.where(colf == lab_col, chunk, 0.0)
            picks = jnp.sum(masked, axis=1)
            r_sc[pl.ds(ci * RG + g, 1), :] = picks.reshape(1, 128)

    key = r_sc[:, :]
    val = jax.nn.sigmoid(w_ref[:, :])
    ssum = jnp.sum(val)

    bit0_i = [1 - ((ii >> s) & 1) for s in range(LOGN)]
    lo_bs = [b == 1 for b in bit0_i]
    ones_i = jnp.full((128, 128), 1, jnp.int32)

    for p in range(1, LOGN + 1):
        up_i = bit0_i[p] if p < LOGN else ones_i
        for s in range(p - 1, -1, -1):
            d = 1 << s
            if d < 128:
                axis, dist = 1, d
            else:
                axis, dist = 0, d >> 7
            lo_i = bit0_i[s]
            lo_b = lo_bs[s]
            kf = pltpu.roll(key, 128 - dist, axis)
            kb = pltpu.roll(key, dist, axis)
            keyB = jnp.where(lo_b, kf, kb)
            vf = pltpu.roll(val, 128 - dist, axis)
            vb = pltpu.roll(val, dist, axis)
            valB = jnp.where(lo_b, vf, vb)
            wm_i = 1 - (lo_i ^ up_i)
            le_i = jnp.where(key <= keyB, 1, 0)
            lt_i = jnp.where(key < keyB, 1, 0)
            cmp_i = jnp.where(lo_b, le_i, lt_i)
            take_b = cmp_i == wm_i
            key = jnp.where(take_b, key, keyB)
            val = jnp.where(take_b, val, valB)

    w = val * (1.0 / (ssum + 1.0))

    ci2 = jax.lax.broadcasted_iota(jnp.int32, (128, 128), 1)
    x = w
    for s in (1, 2, 4, 8, 16, 32, 64):
        sh = pltpu.roll(x, s, 1)
        x = x + jnp.where(ci2 >= s, sh, 0.0)
    row_tot = jnp.sum(w, axis=1, keepdims=True)
    ri1 = jax.lax.broadcasted_iota(jnp.int32, (128, 1), 0)
    y = row_tot
    for s in (1, 2, 4, 8, 16, 32, 64):
        sh = pltpu.roll(y, s, 0)
        y = y + jnp.where(ri1 >= s, sh, 0.0)
    c = x + (y - row_tot)

    resi = c - (1.0 - ALPHA)
    xx = -(resi * resi) * (1.0 / SIGMA)
    m = jnp.max(xx)
    e = jnp.exp(xx - m)
    se = jnp.sum(e)
    num = jnp.sum(key * e)
    out_ref[:, :] = jnp.full((1, 1), num / se, jnp.float32)


def kernel(cal_smx, cal_labels, weights):
    scratch = [pltpu.VMEM((128, 128), jnp.float32)]
    scratch += [pltpu.VMEM((CH, C), jnp.float32) for _ in range(NBUF)]
    scratch += [pltpu.SemaphoreType.DMA for _ in range(NBUF)]

    out = pl.pallas_call(
        _fused_body,
        in_specs=[
            pl.BlockSpec((128, 128), lambda: (0, 0)),
            pl.BlockSpec((128, 128), lambda: (0, 0)),
            pl.BlockSpec(memory_space=pl.ANY),
        ],
        out_specs=pl.BlockSpec((1, 1), lambda: (0, 0)),
        out_shape=jax.ShapeDtypeStruct((1, 1), jnp.float32),
        scratch_shapes=scratch,
    )(cal_labels.astype(jnp.int32).reshape(128, 128),
      weights.reshape(128, 128), cal_smx)

    q = out[0, 0]
    return (q, q)


# speedup vs baseline: 1.0077x; 1.0077x over previous
import jax
import jax.numpy as jnp
from jax.experimental import pallas as pl
from jax.experimental.pallas import tpu as pltpu

K = 16384
C = 1000
ALPHA = 0.1
SIGMA = 0.01

LOGN = 14
CH = 512
NCH = K // CH
NBUF = 4


def _fused_body(lab_ref, w_ref, smx_ref, out_ref, r_sc, *bufs_and_sems):
    bufs = bufs_and_sems[:NBUF]
    sems = bufs_and_sems[NBUF:]

    def dma(ci):
        return pltpu.make_async_copy(
            smx_ref.at[pl.ds(ci * CH, CH), :], bufs[ci % NBUF], sems[ci % NBUF])

    for ci in range(NBUF):
        dma(ci).start()

    ii = jax.lax.broadcasted_iota(jnp.int32, (128, 128), 0) * 128 + \
        jax.lax.broadcasted_iota(jnp.int32, (128, 128), 1)
    eye = jnp.where(
        jax.lax.broadcasted_iota(jnp.int32, (128, 128), 0)
        == jax.lax.broadcasted_iota(jnp.int32, (128, 128), 1), 1.0, 0.0
    ).astype(jnp.float32)
    colf = jax.lax.broadcasted_iota(jnp.int32, (128, C), 1).astype(jnp.float32)

    RG = CH // 128
    for ci in range(NCH):
        dma(ci).wait()
        if ci + NBUF < NCH:
            dma(ci + NBUF).start()
        for g in range(RG):
            lrf = lab_ref[pl.ds(ci * RG + g, 1), :].astype(jnp.float32)
            lab_col = jax.lax.dot_general(
                eye, lrf, (((1,), (1,)), ((), ())),
                preferred_element_type=jnp.float32)
            chunk = bufs[ci % NBUF][pl.ds(g * 128, 128), :]
            masked = jnp.where(colf == lab_col, chunk, 0.0)
            picks = jnp.sum(masked, axis=1)
            r_sc[pl.ds(ci * RG + g, 1), :] = picks.reshape(1, 128)

    key = r_sc[:, :]
    val = jax.nn.sigmoid(w_ref[:, :])
    ssum = jnp.sum(val)

    bit0_i = [1 - ((ii >> s) & 1) for s in range(LOGN)]
    lo_bs = [b == 1 for b in bit0_i]
    ones_i = jnp.full((128, 128), 1, jnp.int32)

    for p in range(1, LOGN + 1):
        up_i = bit0_i[p] if p < LOGN else ones_i
        for s in range(p - 1, -1, -1):
            d = 1 << s
            if d < 128:
                axis, dist = 1, d
            else:
                axis, dist = 0, d >> 7
            lo_i = bit0_i[s]
            lo_b = lo_bs[s]
            kf = pltpu.roll(key, 128 - dist, axis)
            kb = pltpu.roll(key, dist, axis)
            keyB = jnp.where(lo_b, kf, kb)
            vf = pltpu.roll(val, 128 - dist, axis)
            vb = pltpu.roll(val, dist, axis)
            valB = jnp.where(lo_b, vf, vb)
            wm_i = 1 - (lo_i ^ up_i)
            le_i = jnp.where(key <= keyB, 1, 0)
            lt_i = jnp.where(key < keyB, 1, 0)
            cmp_i = jnp.where(lo_b, le_i, lt_i)
            take_b = cmp_i == wm_i
            key = jnp.where(take_b, key, keyB)
            val = jnp.where(take_b, val, valB)

    w = val * (1.0 / (ssum + 1.0))

    ci2 = jax.lax.broadcasted_iota(jnp.int32, (128, 128), 1)
    x = w
    for s in (1, 2, 4, 8, 16, 32, 64):
        sh = pltpu.roll(x, s, 1)
        x = x + jnp.where(ci2 >= s, sh, 0.0)
    row_tot = jnp.sum(w, axis=1, keepdims=True)
    ri1 = jax.lax.broadcasted_iota(jnp.int32, (128, 1), 0)
    y = row_tot
    for s in (1, 2, 4, 8, 16, 32, 64):
        sh = pltpu.roll(y, s, 0)
        y = y + jnp.where(ri1 >= s, sh, 0.0)
    c = x + (y - row_tot)

    resi = c - (1.0 - ALPHA)
    xx = -(resi * resi) * (1.0 / SIGMA)
    m = jnp.max(xx)
    e = jnp.exp(xx - m)
    se = jnp.sum(e)
    num = jnp.sum(key * e)
    out_ref[:, :] = jnp.full((1, 1), num / se, jnp.float32)


def kernel(cal_smx, cal_labels, weights):
    scratch = [pltpu.VMEM((128, 128), jnp.float32)]
    scratch += [pltpu.VMEM((CH, C), jnp.float32) for _ in range(NBUF)]
    scratch += [pltpu.SemaphoreType.DMA for _ in range(NBUF)]

    out = pl.pallas_call(
        _fused_body,
        in_specs=[
            pl.BlockSpec((128, 128), lambda: (0, 0)),
            pl.BlockSpec((128, 128), lambda: (0, 0)),
            pl.BlockSpec(memory_space=pl.ANY),
        ],
        out_specs=pl.BlockSpec((1, 1), lambda: (0, 0)),
        out_shape=jax.ShapeDtypeStruct((1, 1), jnp.float32),
        scratch_shapes=scratch,
    )(cal_labels.astype(jnp.int32).reshape(128, 128),
      weights.reshape(128, 128), cal_smx)

    q = out[0, 0]
    return (q, q)
